# Initial kernel scaffold; baseline (speedup 1.0000x reference)
import jax
import jax.numpy as jnp
from jax.experimental import pallas as pl


def kernel(x, edge_index, batch, params):
    raise NotImplementedError("write your pallas kernel here")


# speedup vs baseline: 2.9162x; 2.9162x over previous
import functools

import jax
import jax.numpy as jnp
from jax import lax
from jax.experimental import pallas as pl
from jax.experimental.pallas import tpu as pltpu
from jax.experimental.pallas import tpu_sc as plsc

N = 10000
E = 320000
G = 64

NC = 2
NS = 16
NW = NC * NS

CHUNK = 128
CH_PER_TILE = 80
E_PAD = CHUNK * CH_PER_TILE * NW
ROWS_SP = 10240
PAD_DST = N


def _make_seg_sum(D):
  e_per_tile = CH_PER_TILE * CHUNK
  mesh = plsc.VectorSubcoreMesh(core_axis_name="c", subcore_axis_name="s")

  @functools.partial(
      pl.kernel,
      mesh=mesh,
      compiler_params=pltpu.CompilerParams(use_tc_tiling_on_sc=(D == 128)),
      out_type=jax.ShapeDtypeStruct((NC, N, D), jnp.float32),
      scratch_types=[
          pltpu.VMEM((CHUNK,), jnp.int32),
          pltpu.VMEM((CHUNK,), jnp.int32),
          pltpu.VMEM((CHUNK, D), jnp.float32),
          pltpu.VMEM_SHARED((ROWS_SP, D), jnp.float32),
          pltpu.SemaphoreType.DMA,
      ],
  )
  def seg_sum(h_hbm, src_hbm, dst_hbm, zeros_hbm, out_hbm,
              src_v, dst_v, rows_v, acc_sh, sem):
    c = lax.axis_index("c")
    s = lax.axis_index("s")
    wid = c * NS + s

    zrows = ROWS_SP // NS
    zbase = s * zrows
    pltpu.sync_copy(zeros_hbm.at[pl.ds(zbase, zrows)],
                    acc_sh.at[pl.ds(zbase, zrows)])
    plsc.subcore_barrier()

    ebase = wid * e_per_tile

    def body(j, carry):
      off = ebase + j * CHUNK
      pltpu.sync_copy(src_hbm.at[pl.ds(off, CHUNK)], src_v)
      pltpu.sync_copy(dst_hbm.at[pl.ds(off, CHUNK)], dst_v)
      pltpu.async_copy(h_hbm.at[src_v], rows_v, sem).wait()
      pltpu.sync_copy(rows_v, acc_sh.at[dst_v], add=True)
      return carry

    lax.fori_loop(0, CH_PER_TILE, body, 0)
    plsc.subcore_barrier()

    orows = 624
    obase = s * orows
    pltpu.sync_copy(acc_sh.at[pl.ds(obase, orows)],
                    out_hbm.at[c].at[pl.ds(obase, orows)])

    @pl.when(s == 0)
    def _tail():
      pltpu.sync_copy(acc_sh.at[pl.ds(NS * orows, N - NS * orows)],
                      out_hbm.at[c].at[pl.ds(NS * orows, N - NS * orows)])

  return seg_sum


_SEG = {128: _make_seg_sum(128), 64: _make_seg_sum(64)}


def _dense_layer(h, agg, W1, b1, g1, be1, W2, b2, gm=None, bm=None):
  mid = gm is not None
  n, _ = h.shape
  dout = W2.shape[1]

  def body(h_ref, a0_ref, a1_ref, w1_ref, b1_ref, g1_ref, be1_ref,
           w2_ref, b2_ref, *rest):
    if mid:
      gm_ref, bm_ref, out_ref = rest
    else:
      (out_ref,) = rest
    z = h_ref[...] + a0_ref[...] + a1_ref[...]
    z = jnp.dot(z, w1_ref[...], preferred_element_type=jnp.float32) + b1_ref[...]
    mu = jnp.mean(z, axis=0, keepdims=True)
    var = jnp.mean((z - mu) * (z - mu), axis=0, keepdims=True)
    z = g1_ref[...] * (z - mu) * lax.rsqrt(var + 1e-5) + be1_ref[...]
    z = jnp.maximum(z, 0.0)
    z = jnp.dot(z, w2_ref[...], preferred_element_type=jnp.float32) + b2_ref[...]
    if mid:
      mu2 = jnp.mean(z, axis=0, keepdims=True)
      var2 = jnp.mean((z - mu2) * (z - mu2), axis=0, keepdims=True)
      z = gm_ref[...] * (z - mu2) * lax.rsqrt(var2 + 1e-5) + bm_ref[...]
      z = jnp.maximum(z, 0.0)
    out_ref[...] = z

  args = [h, agg[0], agg[1], W1, b1.reshape(1, -1), g1.reshape(1, -1),
          be1.reshape(1, -1), W2, b2.reshape(1, -1)]
  if mid:
    args += [gm.reshape(1, -1), bm.reshape(1, -1)]
  return pl.pallas_call(
      body,
      out_shape=jax.ShapeDtypeStruct((n, dout), jnp.float32),
  )(*args)


def _pool(h, batch_row):
  n, dout = h.shape

  def body(h_ref, b_ref, out_ref):
    oh = (lax.broadcasted_iota(jnp.int32, (G, n), 0) == b_ref[...]).astype(
        jnp.float32)
    sums = jnp.dot(oh, h_ref[...], preferred_element_type=jnp.float32)
    cnt = jnp.sum(oh, axis=1, keepdims=True)
    mean = sums / jnp.maximum(cnt, 1.0)
    mx = jnp.max(mean, axis=1, keepdims=True)
    lse = jnp.log(jnp.sum(jnp.exp(mean - mx), axis=1, keepdims=True)) + mx
    out_ref[...] = mean - lse

  return pl.pallas_call(
      body,
      out_shape=jax.ShapeDtypeStruct((G, dout), jnp.float32),
  )(h, batch_row)


def kernel(x, edge_index, batch, params):
  p = list(params)
  layer_p = [p[i * 6:(i + 1) * 6] for i in range(5)]
  norm_p = [p[30 + i * 2:30 + (i + 1) * 2] for i in range(4)]

  pad = E_PAD - E
  src_p = jnp.concatenate([edge_index[0], jnp.zeros((pad,), jnp.int32)])
  dst_p = jnp.concatenate([edge_index[1],
                           jnp.full((pad,), PAD_DST, jnp.int32)])
  zeros = {d: jnp.zeros((ROWS_SP, d), jnp.float32) for d in (128, 64)}

  h = x
  for i in range(5):
    d = h.shape[1]
    agg = _SEG[d](h, src_p, dst_p, zeros[d])
    W1, b1, g1, be1, W2, b2 = layer_p[i]
    if i < 4:
      gm, bm = norm_p[i]
      h = _dense_layer(h, agg, W1, b1, g1, be1, W2, b2, gm, bm)
    else:
      h = _dense_layer(h, agg, W1, b1, g1, be1, W2, b2)

  return _pool(h, batch.reshape(1, N).astype(jnp.int32))


# speedup vs baseline: 5.2591x; 1.8034x over previous
import functools

import jax
import jax.numpy as jnp
from jax import lax
from jax.experimental import pallas as pl
from jax.experimental.pallas import tpu as pltpu
from jax.experimental.pallas import tpu_sc as plsc

N = 10000
E = 320000
G = 64
D = 64

NC = 2
NS = 16
NW = NC * NS

CHUNK = 128
CH_PER_TILE = 80
E_PAD = CHUNK * CH_PER_TILE * NW
ROWS_SP = 10240
PAD_DST = N

NB = 8
DA = NB // 2
NSTEP = CH_PER_TILE


def _make_seg_sum():
  mesh = plsc.VectorSubcoreMesh(core_axis_name="c", subcore_axis_name="s")

  @functools.partial(
      pl.kernel,
      mesh=mesh,
      compiler_params=pltpu.CompilerParams(use_tc_tiling_on_sc=False),
      out_type=jax.ShapeDtypeStruct((NC, N, D), jnp.float32),
      scratch_types=[
          pltpu.VMEM((CH_PER_TILE, CHUNK), jnp.int32),
          pltpu.VMEM((CH_PER_TILE, CHUNK), jnp.int32),
          pltpu.VMEM((NB, CHUNK, D), jnp.float32),
          pltpu.VMEM_SHARED((ROWS_SP, D), jnp.float32),
          pltpu.SemaphoreType.DMA((NB,)),
          pltpu.SemaphoreType.DMA((NB,)),
      ],
  )
  def seg_sum(h_hbm, src_hbm, dst_hbm, zeros_hbm, out_hbm,
              src2d, dst2d, rows, acc_sh, gsem, ssem):
    c = lax.axis_index("c")
    s = lax.axis_index("s")
    wid = c * NS + s

    pltpu.async_copy(src_hbm.at[wid], src2d, gsem.at[0]).wait()
    pltpu.async_copy(dst_hbm.at[wid], dst2d, gsem.at[0]).wait()

    zrows = ROWS_SP // NS
    zbase = s * zrows
    pltpu.sync_copy(zeros_hbm.at[pl.ds(zbase, zrows)],
                    acc_sh.at[pl.ds(zbase, zrows)])
    plsc.subcore_barrier()

    def gather(k, b):
      return pltpu.make_async_copy(h_hbm.at[src2d.at[k]], rows.at[b],
                                   gsem.at[b])

    def scatter(k, b):
      return pltpu.make_async_copy(rows.at[b], acc_sh.at[dst2d.at[k]],
                                   ssem.at[b])

    for b in range(DA):
      pltpu.async_copy(h_hbm.at[src2d.at[b]], rows.at[b], gsem.at[b])

    def body(jj, carry):
      base = jj * NB
      for b in range(NB):
        k = base + b
        gather(k, b).wait()
        pltpu.async_copy(rows.at[b], acc_sh.at[dst2d.at[k]], ssem.at[b],
                         add=True)
        kn = k + DA
        bn = (b + DA) % NB

        @pl.when(jnp.logical_and(kn >= NB, kn < NSTEP))
        def _wait_prev_scatter():
          scatter(kn - NB, bn).wait()

        @pl.when(kn < NSTEP)
        def _fire_ahead():
          pltpu.async_copy(h_hbm.at[src2d.at[kn]], rows.at[bn], gsem.at[bn])
      return carry

    lax.fori_loop(0, NSTEP // NB, body, 0)
    for b in range(NB):
      scatter(NSTEP - NB + b, b).wait()
    plsc.subcore_barrier()

    orows = 624
    obase = s * orows
    pltpu.sync_copy(acc_sh.at[pl.ds(obase, orows)],
                    out_hbm.at[c].at[pl.ds(obase, orows)])

    @pl.when(s == 0)
    def _tail():
      pltpu.sync_copy(acc_sh.at[pl.ds(NS * orows, N - NS * orows)],
                      out_hbm.at[c].at[pl.ds(NS * orows, N - NS * orows)])

  return seg_sum


_SEG = _make_seg_sum()


def _pre(x, W1):
  n = x.shape[0]
  dh = W1.shape[1]

  def body(x_ref, w_ref, out_ref):
    out_ref[...] = jnp.dot(x_ref[...], w_ref[...],
                           preferred_element_type=jnp.float32)

  return pl.pallas_call(
      body, out_shape=jax.ShapeDtypeStruct((n, dh), jnp.float32))(x, W1)


def _dense_layer(u, agg, b1, g1, be1, W2, b2, gm, bm, W1n):
  mid = gm is not None
  n = u.shape[0]
  dout = W1n.shape[1] if W1n is not None else W2.shape[1]

  def body(*refs):
    refs = list(refs)
    u_ref, a0_ref, a1_ref, b1_ref, g1_ref, be1_ref, w2_ref, b2_ref = refs[:8]
    refs = refs[8:]
    if mid:
      gm_ref, bm_ref = refs[:2]
      refs = refs[2:]
    if W1n is not None:
      w1n_ref = refs[0]
      refs = refs[1:]
    out_ref = refs[0]

    z = u_ref[...] + a0_ref[...] + a1_ref[...] + b1_ref[...]
    mu = jnp.mean(z, axis=0, keepdims=True)
    var = jnp.mean((z - mu) * (z - mu), axis=0, keepdims=True)
    z = g1_ref[...] * (z - mu) * lax.rsqrt(var + 1e-5) + be1_ref[...]
    z = jnp.maximum(z, 0.0)
    z = jnp.dot(z, w2_ref[...], preferred_element_type=jnp.float32) + b2_ref[...]
    if mid:
      mu2 = jnp.mean(z, axis=0, keepdims=True)
      var2 = jnp.mean((z - mu2) * (z - mu2), axis=0, keepdims=True)
      z = gm_ref[...] * (z - mu2) * lax.rsqrt(var2 + 1e-5) + bm_ref[...]
      z = jnp.maximum(z, 0.0)
    if W1n is not None:
      z = jnp.dot(z, w1n_ref[...], preferred_element_type=jnp.float32)
    out_ref[...] = z

  args = [u, agg[0], agg[1], b1.reshape(1, -1), g1.reshape(1, -1),
          be1.reshape(1, -1), W2, b2.reshape(1, -1)]
  if mid:
    args += [gm.reshape(1, -1), bm.reshape(1, -1)]
  if W1n is not None:
    args += [W1n]
  return pl.pallas_call(
      body,
      out_shape=jax.ShapeDtypeStruct((n, dout), jnp.float32),
  )(*args)


def _final_layer(h, agg, W1, b1, g1, be1, W2, b2):
  n = h.shape[0]
  dout = W2.shape[1]

  def body(h_ref, a0_ref, a1_ref, w1_ref, b1_ref, g1_ref, be1_ref,
           w2_ref, b2_ref, out_ref):
    z = h_ref[...] + a0_ref[...] + a1_ref[...]
    z = jnp.dot(z, w1_ref[...], preferred_element_type=jnp.float32) + b1_ref[...]
    mu = jnp.mean(z, axis=0, keepdims=True)
    var = jnp.mean((z - mu) * (z - mu), axis=0, keepdims=True)
    z = g1_ref[...] * (z - mu) * lax.rsqrt(var + 1e-5) + be1_ref[...]
    z = jnp.maximum(z, 0.0)
    z = jnp.dot(z, w2_ref[...], preferred_element_type=jnp.float32) + b2_ref[...]
    out_ref[...] = z

  return pl.pallas_call(
      body,
      out_shape=jax.ShapeDtypeStruct((n, dout), jnp.float32),
  )(h, agg[0], agg[1], W1, b1.reshape(1, -1), g1.reshape(1, -1),
    be1.reshape(1, -1), W2, b2.reshape(1, -1))


def _pool(h, batch_row):
  n, dout = h.shape

  def body(h_ref, b_ref, out_ref):
    oh = (lax.broadcasted_iota(jnp.int32, (G, n), 0) == b_ref[...]).astype(
        jnp.float32)
    sums = jnp.dot(oh, h_ref[...], preferred_element_type=jnp.float32)
    cnt = jnp.sum(oh, axis=1, keepdims=True)
    mean = sums / jnp.maximum(cnt, 1.0)
    mx = jnp.max(mean, axis=1, keepdims=True)
    lse = jnp.log(jnp.sum(jnp.exp(mean - mx), axis=1, keepdims=True)) + mx
    out_ref[...] = mean - lse

  return pl.pallas_call(
      body,
      out_shape=jax.ShapeDtypeStruct((G, dout), jnp.float32),
  )(h, batch_row)


def kernel(x, edge_index, batch, params):
  p = list(params)
  layer_p = [p[i * 6:(i + 1) * 6] for i in range(5)]
  norm_p = [p[30 + i * 2:30 + (i + 1) * 2] for i in range(4)]

  pad = E_PAD - E
  src_p = jnp.concatenate([edge_index[0], jnp.zeros((pad,), jnp.int32)])
  src_p = src_p.reshape(NW, CH_PER_TILE, CHUNK)
  dst_p = jnp.concatenate([edge_index[1],
                           jnp.full((pad,), PAD_DST, jnp.int32)])
  dst_p = dst_p.reshape(NW, CH_PER_TILE, CHUNK)
  zeros = jnp.zeros((ROWS_SP, D), jnp.float32)

  u = _pre(x, layer_p[0][0])
  for i in range(4):
    _, b1, g1, be1, W2, b2 = layer_p[i]
    gm, bm = norm_p[i]
    agg = _SEG(u, src_p, dst_p, zeros)
    W1n = layer_p[i + 1][0] if i < 3 else None
    u = _dense_layer(u, agg, b1, g1, be1, W2, b2, gm, bm, W1n)

  W1, b1, g1, be1, W2, b2 = layer_p[4]
  agg = _SEG(u, src_p, dst_p, zeros)
  h5 = _final_layer(u, agg, W1, b1, g1, be1, W2, b2)

  return _pool(h5, batch.reshape(1, N).astype(jnp.int32))


# speedup vs baseline: 12.0664x; 2.2944x over previous
import functools

import jax
import jax.numpy as jnp
from jax import lax
from jax.experimental import pallas as pl
from jax.experimental.pallas import tpu as pltpu
from jax.experimental.pallas import tpu_sc as plsc

N = 10000
E = 320000
G = 64
D = 64

NC = 2
NS = 16
NW = NC * NS

CHUNK = 64
CH_PER_TILE = 160
E_PAD = CHUNK * CH_PER_TILE * NW
ROWS_SP = 10240
PAD_DST = N

NB = 5
DA = 2
NSTEP = CH_PER_TILE


def _make_seg_sum():
  mesh = plsc.VectorSubcoreMesh(core_axis_name="c", subcore_axis_name="s")

  @functools.partial(
      pl.kernel,
      mesh=mesh,
      compiler_params=pltpu.CompilerParams(use_tc_tiling_on_sc=False),
      out_type=jax.ShapeDtypeStruct((NC, N, D), jnp.float32),
      scratch_types=[
          pltpu.VMEM((CH_PER_TILE, CHUNK), jnp.int32),
          pltpu.VMEM((CH_PER_TILE, CHUNK), jnp.int32),
          pltpu.VMEM((NB, CHUNK, D), jnp.float32),
          pltpu.VMEM_SHARED((N, D), jnp.float32),
          pltpu.VMEM_SHARED((ROWS_SP, D), jnp.float32),
          pltpu.SemaphoreType.DMA((NB,)),
          pltpu.SemaphoreType.DMA((NB,)),
      ],
  )
  def seg_sum(h_hbm, src_hbm, dst_hbm, zeros_hbm, out_hbm,
              src2d, dst2d, rows, h_sh, acc_sh, gsem, ssem):
    c = lax.axis_index("c")
    s = lax.axis_index("s")
    wid = c * NS + s

    pltpu.async_copy(src_hbm.at[wid], src2d, gsem.at[0]).wait()
    pltpu.async_copy(dst_hbm.at[wid], dst2d, gsem.at[0]).wait()

    hrows = N // NS
    hbase = s * hrows
    pltpu.sync_copy(h_hbm.at[pl.ds(hbase, hrows)],
                    h_sh.at[pl.ds(hbase, hrows)])
    zrows = ROWS_SP // NS
    zbase = s * zrows
    pltpu.sync_copy(zeros_hbm.at[pl.ds(zbase, zrows)],
                    acc_sh.at[pl.ds(zbase, zrows)])
    plsc.subcore_barrier()

    def gather(k, b):
      return pltpu.make_async_copy(h_sh.at[src2d.at[k]], rows.at[b],
                                   gsem.at[b])

    def scatter(k, b):
      return pltpu.make_async_copy(rows.at[b], acc_sh.at[dst2d.at[k]],
                                   ssem.at[b])

    for b in range(DA):
      pltpu.async_copy(h_sh.at[src2d.at[b]], rows.at[b], gsem.at[b])

    def body(jj, carry):
      base = jj * NB
      for b in range(NB):
        k = base + b
        gather(k, b).wait()
        pltpu.async_copy(rows.at[b], acc_sh.at[dst2d.at[k]], ssem.at[b],
                         add=True)
        kn = k + DA
        bn = (b + DA) % NB

        @pl.when(jnp.logical_and(kn >= NB, kn < NSTEP))
        def _wait_prev_scatter():
          scatter(kn - NB, bn).wait()

        @pl.when(kn < NSTEP)
        def _fire_ahead():
          pltpu.async_copy(h_sh.at[src2d.at[kn]], rows.at[bn], gsem.at[bn])
      return carry

    lax.fori_loop(0, NSTEP // NB, body, 0)
    for b in range(NB):
      scatter(NSTEP - NB + b, b).wait()
    plsc.subcore_barrier()

    orows = 624
    obase = s * orows
    pltpu.sync_copy(acc_sh.at[pl.ds(obase, orows)],
                    out_hbm.at[c].at[pl.ds(obase, orows)])

    @pl.when(s == 0)
    def _tail():
      pltpu.sync_copy(acc_sh.at[pl.ds(NS * orows, N - NS * orows)],
                      out_hbm.at[c].at[pl.ds(NS * orows, N - NS * orows)])

  return seg_sum


_SEG = _make_seg_sum()


def _pre(x, W1):
  n = x.shape[0]
  dh = W1.shape[1]

  def body(x_ref, w_ref, out_ref):
    out_ref[...] = jnp.dot(x_ref[...], w_ref[...],
                           preferred_element_type=jnp.float32)

  return pl.pallas_call(
      body, out_shape=jax.ShapeDtypeStruct((n, dh), jnp.float32))(x, W1)


def _dense_layer(u, agg, b1, g1, be1, W2, b2, gm, bm, W1n):
  mid = gm is not None
  n = u.shape[0]
  dout = W1n.shape[1] if W1n is not None else W2.shape[1]

  def body(*refs):
    refs = list(refs)
    u_ref, a0_ref, a1_ref, b1_ref, g1_ref, be1_ref, w2_ref, b2_ref = refs[:8]
    refs = refs[8:]
    if mid:
      gm_ref, bm_ref = refs[:2]
      refs = refs[2:]
    if W1n is not None:
      w1n_ref = refs[0]
      refs = refs[1:]
    out_ref = refs[0]

    z = u_ref[...] + a0_ref[...] + a1_ref[...] + b1_ref[...]
    mu = jnp.mean(z, axis=0, keepdims=True)
    var = jnp.mean((z - mu) * (z - mu), axis=0, keepdims=True)
    z = g1_ref[...] * (z - mu) * lax.rsqrt(var + 1e-5) + be1_ref[...]
    z = jnp.maximum(z, 0.0)
    z = jnp.dot(z, w2_ref[...], preferred_element_type=jnp.float32) + b2_ref[...]
    if mid:
      mu2 = jnp.mean(z, axis=0, keepdims=True)
      var2 = jnp.mean((z - mu2) * (z - mu2), axis=0, keepdims=True)
      z = gm_ref[...] * (z - mu2) * lax.rsqrt(var2 + 1e-5) + bm_ref[...]
      z = jnp.maximum(z, 0.0)
    if W1n is not None:
      z = jnp.dot(z, w1n_ref[...], preferred_element_type=jnp.float32)
    out_ref[...] = z

  args = [u, agg[0], agg[1], b1.reshape(1, -1), g1.reshape(1, -1),
          be1.reshape(1, -1), W2, b2.reshape(1, -1)]
  if mid:
    args += [gm.reshape(1, -1), bm.reshape(1, -1)]
  if W1n is not None:
    args += [W1n]
  return pl.pallas_call(
      body,
      out_shape=jax.ShapeDtypeStruct((n, dout), jnp.float32),
  )(*args)


def _final_layer(h, agg, W1, b1, g1, be1, W2, b2):
  n = h.shape[0]
  dout = W2.shape[1]

  def body(h_ref, a0_ref, a1_ref, w1_ref, b1_ref, g1_ref, be1_ref,
           w2_ref, b2_ref, out_ref):
    z = h_ref[...] + a0_ref[...] + a1_ref[...]
    z = jnp.dot(z, w1_ref[...], preferred_element_type=jnp.float32) + b1_ref[...]
    mu = jnp.mean(z, axis=0, keepdims=True)
    var = jnp.mean((z - mu) * (z - mu), axis=0, keepdims=True)
    z = g1_ref[...] * (z - mu) * lax.rsqrt(var + 1e-5) + be1_ref[...]
    z = jnp.maximum(z, 0.0)
    z = jnp.dot(z, w2_ref[...], preferred_element_type=jnp.float32) + b2_ref[...]
    out_ref[...] = z

  return pl.pallas_call(
      body,
      out_shape=jax.ShapeDtypeStruct((n, dout), jnp.float32),
  )(h, agg[0], agg[1], W1, b1.reshape(1, -1), g1.reshape(1, -1),
    be1.reshape(1, -1), W2, b2.reshape(1, -1))


def _pool(h, batch_row):
  n, dout = h.shape

  def body(h_ref, b_ref, out_ref):
    oh = (lax.broadcasted_iota(jnp.int32, (G, n), 0) == b_ref[...]).astype(
        jnp.float32)
    sums = jnp.dot(oh, h_ref[...], preferred_element_type=jnp.float32)
    cnt = jnp.sum(oh, axis=1, keepdims=True)
    mean = sums / jnp.maximum(cnt, 1.0)
    mx = jnp.max(mean, axis=1, keepdims=True)
    lse = jnp.log(jnp.sum(jnp.exp(mean - mx), axis=1, keepdims=True)) + mx
    out_ref[...] = mean - lse

  return pl.pallas_call(
      body,
      out_shape=jax.ShapeDtypeStruct((G, dout), jnp.float32),
  )(h, batch_row)


def kernel(x, edge_index, batch, params):
  p = list(params)
  layer_p = [p[i * 6:(i + 1) * 6] for i in range(5)]
  norm_p = [p[30 + i * 2:30 + (i + 1) * 2] for i in range(4)]

  pad = E_PAD - E
  src_p = jnp.concatenate([edge_index[0], jnp.zeros((pad,), jnp.int32)])
  src_p = src_p.reshape(NW, CH_PER_TILE, CHUNK)
  dst_p = jnp.concatenate([edge_index[1],
                           jnp.full((pad,), PAD_DST, jnp.int32)])
  dst_p = dst_p.reshape(NW, CH_PER_TILE, CHUNK)
  zeros = jnp.zeros((ROWS_SP, D), jnp.float32)

  u = _pre(x, layer_p[0][0])
  for i in range(4):
    _, b1, g1, be1, W2, b2 = layer_p[i]
    gm, bm = norm_p[i]
    agg = _SEG(u, src_p, dst_p, zeros)
    W1n = layer_p[i + 1][0] if i < 3 else None
    u = _dense_layer(u, agg, b1, g1, be1, W2, b2, gm, bm, W1n)

  W1, b1, g1, be1, W2, b2 = layer_p[4]
  agg = _SEG(u, src_p, dst_p, zeros)
  h5 = _final_layer(u, agg, W1, b1, g1, be1, W2, b2)

  return _pool(h5, batch.reshape(1, N).astype(jnp.int32))


# speedup vs baseline: 12.9196x; 1.0707x over previous
import functools

import jax
import jax.numpy as jnp
from jax import lax
from jax.experimental import pallas as pl
from jax.experimental.pallas import tpu as pltpu
from jax.experimental.pallas import tpu_sc as plsc

N = 10000
E = 320000
G = 64
D = 64

NC = 2
NS = 16
NW = NC * NS

CHUNK = 64
CH_PER_TILE = 160
E_PAD = CHUNK * CH_PER_TILE * NW
ROWS_SP = 10240
PAD_DST = N

NB = 5
DA = 2
NSTEP = CH_PER_TILE


def _make_seg_sum():
  mesh = plsc.VectorSubcoreMesh(core_axis_name="c", subcore_axis_name="s")

  @functools.partial(
      pl.kernel,
      mesh=mesh,
      compiler_params=pltpu.CompilerParams(use_tc_tiling_on_sc=False),
      out_type=jax.ShapeDtypeStruct((NC, N, D), jnp.float32),
      scratch_types=[
          pltpu.VMEM((CH_PER_TILE, CHUNK), jnp.int32),
          pltpu.VMEM((CH_PER_TILE, CHUNK), jnp.int32),
          pltpu.VMEM((NB, CHUNK, D), jnp.float32),
          pltpu.VMEM_SHARED((N, D), jnp.float32),
          pltpu.VMEM_SHARED((ROWS_SP, D), jnp.float32),
          pltpu.SemaphoreType.DMA((NB,)),
          pltpu.SemaphoreType.DMA((NB,)),
      ],
  )
  def seg_sum(h_hbm, src_hbm, dst_hbm, zeros_hbm, out_hbm,
              src2d, dst2d, rows, h_sh, acc_sh, gsem, ssem):
    c = lax.axis_index("c")
    s = lax.axis_index("s")
    wid = c * NS + s

    pltpu.async_copy(src_hbm.at[wid], src2d, gsem.at[0]).wait()
    pltpu.async_copy(dst_hbm.at[wid], dst2d, gsem.at[0]).wait()

    hrows = 624
    hbase = s * hrows
    pltpu.sync_copy(h_hbm.at[pl.ds(hbase, hrows)],
                    h_sh.at[pl.ds(hbase, hrows)])

    @pl.when(s == 0)
    def _h_tail():
      pltpu.sync_copy(h_hbm.at[pl.ds(NS * hrows, N - NS * hrows)],
                      h_sh.at[pl.ds(NS * hrows, N - NS * hrows)])
    zrows = ROWS_SP // NS
    zbase = s * zrows
    pltpu.sync_copy(zeros_hbm.at[pl.ds(zbase, zrows)],
                    acc_sh.at[pl.ds(zbase, zrows)])
    plsc.subcore_barrier()

    def gather(k, b):
      return pltpu.make_async_copy(h_sh.at[src2d.at[k]], rows.at[b],
                                   gsem.at[b])

    def scatter(k, b):
      return pltpu.make_async_copy(rows.at[b], acc_sh.at[dst2d.at[k]],
                                   ssem.at[b])

    for b in range(DA):
      pltpu.async_copy(h_sh.at[src2d.at[b]], rows.at[b], gsem.at[b])

    def body(jj, carry):
      base = jj * NB
      for b in range(NB):
        k = base + b
        gather(k, b).wait()
        pltpu.async_copy(rows.at[b], acc_sh.at[dst2d.at[k]], ssem.at[b],
                         add=True)
        kn = k + DA
        bn = (b + DA) % NB

        @pl.when(jnp.logical_and(kn >= NB, kn < NSTEP))
        def _wait_prev_scatter():
          scatter(kn - NB, bn).wait()

        @pl.when(kn < NSTEP)
        def _fire_ahead():
          pltpu.async_copy(h_sh.at[src2d.at[kn]], rows.at[bn], gsem.at[bn])
      return carry

    lax.fori_loop(0, NSTEP // NB, body, 0)
    for b in range(NB):
      scatter(NSTEP - NB + b, b).wait()
    plsc.subcore_barrier()

    orows = 624
    obase = s * orows
    pltpu.sync_copy(acc_sh.at[pl.ds(obase, orows)],
                    out_hbm.at[c].at[pl.ds(obase, orows)])

    @pl.when(s == 0)
    def _tail():
      pltpu.sync_copy(acc_sh.at[pl.ds(NS * orows, N - NS * orows)],
                      out_hbm.at[c].at[pl.ds(NS * orows, N - NS * orows)])

  return seg_sum


_SEG = _make_seg_sum()


def _pre(x, W1):
  n = x.shape[0]
  dh = W1.shape[1]

  def body(x_ref, w_ref, out_ref):
    out_ref[...] = jnp.dot(x_ref[...], w_ref[...],
                           preferred_element_type=jnp.float32)

  return pl.pallas_call(
      body, out_shape=jax.ShapeDtypeStruct((n, dh), jnp.float32))(x, W1)


def _dense_layer(u, agg, b1, g1, be1, W2, b2, gm, bm, W1n):
  mid = gm is not None
  n = u.shape[0]
  dout = W1n.shape[1] if W1n is not None else W2.shape[1]

  def body(*refs):
    refs = list(refs)
    u_ref, agg_ref, b1_ref, g1_ref, be1_ref, w2_ref, b2_ref = refs[:7]
    refs = refs[7:]
    if mid:
      gm_ref, bm_ref = refs[:2]
      refs = refs[2:]
    if W1n is not None:
      w1n_ref = refs[0]
      refs = refs[1:]
    out_ref = refs[0]

    z = u_ref[...] + agg_ref[0] + agg_ref[1] + b1_ref[...]
    mu = jnp.mean(z, axis=0, keepdims=True)
    var = jnp.mean((z - mu) * (z - mu), axis=0, keepdims=True)
    z = g1_ref[...] * (z - mu) * lax.rsqrt(var + 1e-5) + be1_ref[...]
    z = jnp.maximum(z, 0.0)
    z = jnp.dot(z, w2_ref[...], preferred_element_type=jnp.float32) + b2_ref[...]
    if mid:
      mu2 = jnp.mean(z, axis=0, keepdims=True)
      var2 = jnp.mean((z - mu2) * (z - mu2), axis=0, keepdims=True)
      z = gm_ref[...] * (z - mu2) * lax.rsqrt(var2 + 1e-5) + bm_ref[...]
      z = jnp.maximum(z, 0.0)
    if W1n is not None:
      z = jnp.dot(z, w1n_ref[...], preferred_element_type=jnp.float32)
    out_ref[...] = z

  args = [u, agg, b1.reshape(1, -1), g1.reshape(1, -1),
          be1.reshape(1, -1), W2, b2.reshape(1, -1)]
  if mid:
    args += [gm.reshape(1, -1), bm.reshape(1, -1)]
  if W1n is not None:
    args += [W1n]
  return pl.pallas_call(
      body,
      out_shape=jax.ShapeDtypeStruct((n, dout), jnp.float32),
  )(*args)


def _final_layer(h, agg, W1, b1, g1, be1, W2, b2, batch_row):
  n = h.shape[0]
  dout = W2.shape[1]

  def body(h_ref, agg_ref, w1_ref, b1_ref, g1_ref, be1_ref,
           w2_ref, b2_ref, bt_ref, out_ref):
    z = h_ref[...] + agg_ref[0] + agg_ref[1]
    z = jnp.dot(z, w1_ref[...], preferred_element_type=jnp.float32) + b1_ref[...]
    mu = jnp.mean(z, axis=0, keepdims=True)
    var = jnp.mean((z - mu) * (z - mu), axis=0, keepdims=True)
    z = g1_ref[...] * (z - mu) * lax.rsqrt(var + 1e-5) + be1_ref[...]
    z = jnp.maximum(z, 0.0)
    z = jnp.dot(z, w2_ref[...], preferred_element_type=jnp.float32) + b2_ref[...]
    oh = (lax.broadcasted_iota(jnp.int32, (G, n), 0) == bt_ref[...]).astype(
        jnp.float32)
    sums = jnp.dot(oh, z, preferred_element_type=jnp.float32)
    cnt = jnp.sum(oh, axis=1, keepdims=True)
    mean = sums / jnp.maximum(cnt, 1.0)
    mx = jnp.max(mean, axis=1, keepdims=True)
    lse = jnp.log(jnp.sum(jnp.exp(mean - mx), axis=1, keepdims=True)) + mx
    out_ref[...] = mean - lse

  return pl.pallas_call(
      body,
      out_shape=jax.ShapeDtypeStruct((G, dout), jnp.float32),
  )(h, agg, W1, b1.reshape(1, -1), g1.reshape(1, -1),
    be1.reshape(1, -1), W2, b2.reshape(1, -1), batch_row)


def kernel(x, edge_index, batch, params):
  p = list(params)
  layer_p = [p[i * 6:(i + 1) * 6] for i in range(5)]
  norm_p = [p[30 + i * 2:30 + (i + 1) * 2] for i in range(4)]

  pad = E_PAD - E
  src_p = jnp.concatenate([edge_index[0], jnp.zeros((pad,), jnp.int32)])
  src_p = src_p.reshape(NW, CH_PER_TILE, CHUNK)
  dst_p = jnp.concatenate([edge_index[1],
                           jnp.full((pad,), PAD_DST, jnp.int32)])
  dst_p = dst_p.reshape(NW, CH_PER_TILE, CHUNK)
  zeros = jnp.zeros((ROWS_SP, D), jnp.float32)

  u = _pre(x, layer_p[0][0])
  for i in range(4):
    _, b1, g1, be1, W2, b2 = layer_p[i]
    gm, bm = norm_p[i]
    agg = _SEG(u, src_p, dst_p, zeros)
    W1n = layer_p[i + 1][0] if i < 3 else None
    u = _dense_layer(u, agg, b1, g1, be1, W2, b2, gm, bm, W1n)

  W1, b1, g1, be1, W2, b2 = layer_p[4]
  agg = _SEG(u, src_p, dst_p, zeros)
  return _final_layer(u, agg, W1, b1, g1, be1, W2, b2,
                      batch.reshape(1, N).astype(jnp.int32))


# speedup vs baseline: 15.7386x; 1.2182x over previous
import functools

import jax
import jax.numpy as jnp
from jax import lax
from jax.experimental import pallas as pl
from jax.experimental.pallas import tpu as pltpu
from jax.experimental.pallas import tpu_sc as plsc

N = 10000
E = 320000
G = 64
D = 64

NC = 2
NS = 16
NW = NC * NS

CHUNK = 64
CH_PER_TILE = 160
E_PAD = CHUNK * CH_PER_TILE * NW
ROWS_SP = 10240
PAD_DST = N

NB = 5
DA = 2
NSTEP = CH_PER_TILE


def _make_seg_sum():
  mesh = plsc.VectorSubcoreMesh(core_axis_name="c", subcore_axis_name="s")

  @functools.partial(
      pl.kernel,
      mesh=mesh,
      compiler_params=pltpu.CompilerParams(use_tc_tiling_on_sc=False),
      out_type=jax.ShapeDtypeStruct((NC, N, D), jnp.float32),
      scratch_types=[
          pltpu.VMEM((CH_PER_TILE, CHUNK), jnp.int32),
          pltpu.VMEM((CH_PER_TILE, CHUNK), jnp.int32),
          pltpu.VMEM((NB, CHUNK, D), jnp.float32),
          pltpu.VMEM_SHARED((N, D), jnp.float32),
          pltpu.VMEM_SHARED((ROWS_SP, D), jnp.float32),
          pltpu.SemaphoreType.DMA((NB,)),
          pltpu.SemaphoreType.DMA((NB,)),
      ],
  )
  def seg_sum(h_hbm, src_hbm, dst_hbm, zeros_hbm, out_hbm,
              src2d, dst2d, rows, h_sh, acc_sh, gsem, ssem):
    c = lax.axis_index("c")
    s = lax.axis_index("s")
    wid = c * NS + s

    pltpu.async_copy(src_hbm.at[wid], src2d, gsem.at[0]).wait()
    pltpu.async_copy(dst_hbm.at[wid], dst2d, gsem.at[0]).wait()

    hrows = 624
    hbase = s * hrows
    pltpu.sync_copy(h_hbm.at[pl.ds(hbase, hrows)],
                    h_sh.at[pl.ds(hbase, hrows)])

    @pl.when(s == 0)
    def _h_tail():
      pltpu.sync_copy(h_hbm.at[pl.ds(NS * hrows, N - NS * hrows)],
                      h_sh.at[pl.ds(NS * hrows, N - NS * hrows)])
    zrows = ROWS_SP // NS
    zbase = s * zrows
    pltpu.sync_copy(zeros_hbm.at[pl.ds(zbase, zrows)],
                    acc_sh.at[pl.ds(zbase, zrows)])
    plsc.subcore_barrier()

    def gather(k, b):
      return pltpu.make_async_copy(h_sh.at[src2d.at[k]], rows.at[b],
                                   gsem.at[b])

    def scatter(k, b):
      return pltpu.make_async_copy(rows.at[b], acc_sh.at[dst2d.at[k]],
                                   ssem.at[b])

    for b in range(DA):
      pltpu.async_copy(h_sh.at[src2d.at[b]], rows.at[b], gsem.at[b])

    def body(jj, carry):
      base = jj * NB
      for b in range(NB):
        k = base + b
        gather(k, b).wait()
        pltpu.async_copy(rows.at[b], acc_sh.at[dst2d.at[k]], ssem.at[b],
                         add=True)
        kn = k + DA
        bn = (b + DA) % NB

        @pl.when(jnp.logical_and(kn >= NB, kn < NSTEP))
        def _wait_prev_scatter():
          scatter(kn - NB, bn).wait()

        @pl.when(kn < NSTEP)
        def _fire_ahead():
          pltpu.async_copy(h_sh.at[src2d.at[kn]], rows.at[bn], gsem.at[bn])
      return carry

    lax.fori_loop(0, NSTEP // NB, body, 0)
    for b in range(NB):
      scatter(NSTEP - NB + b, b).wait()
    plsc.subcore_barrier()

    orows = 624
    obase = s * orows
    pltpu.sync_copy(acc_sh.at[pl.ds(obase, orows)],
                    out_hbm.at[c].at[pl.ds(obase, orows)])

    @pl.when(s == 0)
    def _tail():
      pltpu.sync_copy(acc_sh.at[pl.ds(NS * orows, N - NS * orows)],
                      out_hbm.at[c].at[pl.ds(NS * orows, N - NS * orows)])

  return seg_sum


_SEG = _make_seg_sum()


N2 = N // 2


def _bd(W):
  d1, d2 = W.shape
  z = jnp.zeros((d1, d2), W.dtype)
  return jnp.concatenate([jnp.concatenate([W, z], axis=1),
                          jnp.concatenate([z, W], axis=1)], axis=0)


def _dup(v):
  return jnp.concatenate([v, v]).reshape(1, -1)


def _bn_packed(z, g, b, half):
  m = jnp.mean(z, axis=0, keepdims=True)
  q = jnp.mean(z * z, axis=0, keepdims=True)
  mf = (m[:, :half] + m[:, half:]) * 0.5
  qf = (q[:, :half] + q[:, half:]) * 0.5
  var = jnp.maximum(qf - mf * mf, 0.0)
  mu_p = jnp.concatenate([mf, mf], axis=1)
  sd_p = jnp.concatenate([lax.rsqrt(var + 1e-5)] * 2, axis=1)
  return g * (z - mu_p) * sd_p + b


def _pre(x, W1):
  dh = W1.shape[1]

  def body(x_ref, w_ref, out_ref):
    out_ref[...] = jnp.dot(x_ref[...], w_ref[...],
                           preferred_element_type=jnp.float32)

  return pl.pallas_call(
      body, out_shape=jax.ShapeDtypeStruct((N, dh), jnp.float32))(x, W1)


def _dense_layer(u2, agg2, b1d, g1d, be1d, W2d, b2d, gmd, bmd, W1nd):
  def body(u_ref, agg_ref, b1_ref, g1_ref, be1_ref, w2_ref, b2_ref,
           gm_ref, bm_ref, w1n_ref, out_ref):
    z = u_ref[...] + agg_ref[0] + agg_ref[1] + b1_ref[...]
    z = _bn_packed(z, g1_ref[...], be1_ref[...], D)
    z = jnp.maximum(z, 0.0)
    z = jnp.dot(z, w2_ref[...], preferred_element_type=jnp.float32) + b2_ref[...]
    z = _bn_packed(z, gm_ref[...], bm_ref[...], D)
    z = jnp.maximum(z, 0.0)
    z = jnp.dot(z, w1n_ref[...], preferred_element_type=jnp.float32)
    out_ref[...] = z

  return pl.pallas_call(
      body,
      out_shape=jax.ShapeDtypeStruct((N2, 2 * D), jnp.float32),
  )(u2, agg2, b1d, g1d, be1d, W2d, b2d, gmd, bmd, W1nd)


def _final_layer(h2, agg2, W1d, b1d, g1d, be1d, W2d, b2d, bt_e, bt_o):
  dout = W2d.shape[1] // 2

  def body(h_ref, agg_ref, w1_ref, b1_ref, g1_ref, be1_ref,
           w2_ref, b2_ref, bte_ref, bto_ref, out_ref):
    z = h_ref[...] + agg_ref[0] + agg_ref[1]
    z = jnp.dot(z, w1_ref[...], preferred_element_type=jnp.float32) + b1_ref[...]
    z = _bn_packed(z, g1_ref[...], be1_ref[...], dout)
    z = jnp.maximum(z, 0.0)
    z = jnp.dot(z, w2_ref[...], preferred_element_type=jnp.float32) + b2_ref[...]
    oh_e = (lax.broadcasted_iota(jnp.int32, (G, N2), 0) ==
            bte_ref[...]).astype(jnp.float32)
    oh_o = (lax.broadcasted_iota(jnp.int32, (G, N2), 0) ==
            bto_ref[...]).astype(jnp.float32)
    se = jnp.dot(oh_e, z, preferred_element_type=jnp.float32)
    so = jnp.dot(oh_o, z, preferred_element_type=jnp.float32)
    sums = se[:, :dout] + so[:, dout:]
    cnt = jnp.sum(oh_e + oh_o, axis=1, keepdims=True)
    mean = sums / jnp.maximum(cnt, 1.0)
    mx = jnp.max(mean, axis=1, keepdims=True)
    lse = jnp.log(jnp.sum(jnp.exp(mean - mx), axis=1, keepdims=True)) + mx
    out_ref[...] = mean - lse

  return pl.pallas_call(
      body,
      out_shape=jax.ShapeDtypeStruct((G, dout), jnp.float32),
  )(h2, agg2, W1d, b1d, g1d, be1d, W2d, b2d, bt_e, bt_o)


def kernel(x, edge_index, batch, params):
  p = list(params)
  layer_p = [p[i * 6:(i + 1) * 6] for i in range(5)]
  norm_p = [p[30 + i * 2:30 + (i + 1) * 2] for i in range(4)]

  pad = E_PAD - E
  src_p = jnp.concatenate([edge_index[0], jnp.zeros((pad,), jnp.int32)])
  src_p = src_p.reshape(NW, CH_PER_TILE, CHUNK)
  dst_p = jnp.concatenate([edge_index[1],
                           jnp.full((pad,), PAD_DST, jnp.int32)])
  dst_p = dst_p.reshape(NW, CH_PER_TILE, CHUNK)
  zeros = jnp.zeros((ROWS_SP, D), jnp.float32)

  u2 = _pre(x, layer_p[0][0]).reshape(N2, 2 * D)
  for i in range(4):
    _, b1, g1, be1, W2, b2 = layer_p[i]
    gm, bm = norm_p[i]
    agg = _SEG(u2.reshape(N, D), src_p, dst_p, zeros)
    agg2 = agg.reshape(2, N2, 2 * D)
    if i < 3:
      W1nd = _bd(layer_p[i + 1][0])
    else:
      W1nd = jnp.eye(2 * D, dtype=jnp.float32)
    u2 = _dense_layer(u2, agg2, _dup(b1), _dup(g1), _dup(be1), _bd(W2),
                      _dup(b2), _dup(gm), _dup(bm), W1nd)

  W1, b1, g1, be1, W2, b2 = layer_p[4]
  agg = _SEG(u2.reshape(N, D), src_p, dst_p, zeros)
  agg2 = agg.reshape(2, N2, 2 * D)
  bt = batch.astype(jnp.int32)
  return _final_layer(u2, agg2, _bd(W1), _dup(b1), _dup(g1), _dup(be1),
                      _bd(W2), _dup(b2),
                      bt[0::2].reshape(1, N2), bt[1::2].reshape(1, N2))


# speedup vs baseline: 16.0172x; 1.0177x over previous
import functools

import jax
import jax.numpy as jnp
from jax import lax
from jax.experimental import pallas as pl
from jax.experimental.pallas import tpu as pltpu
from jax.experimental.pallas import tpu_sc as plsc

N = 10000
E = 320000
G = 64
D = 64

NC = 2
NS = 16
NW = NC * NS

CHUNK = 50
CH_PER_TILE = 200
ROWS_SP = 10240

NB = 8
DA = 4
NSTEP = CH_PER_TILE


def _make_seg_sum():
  mesh = plsc.VectorSubcoreMesh(core_axis_name="c", subcore_axis_name="s")

  @functools.partial(
      pl.kernel,
      mesh=mesh,
      compiler_params=pltpu.CompilerParams(use_tc_tiling_on_sc=False),
      out_type=jax.ShapeDtypeStruct((NC, N, D), jnp.float32),
      scratch_types=[
          pltpu.VMEM((CH_PER_TILE, CHUNK), jnp.int32),
          pltpu.VMEM((CH_PER_TILE, CHUNK), jnp.int32),
          pltpu.VMEM((NB, CHUNK, D), jnp.float32),
          pltpu.VMEM_SHARED((N, D), jnp.float32),
          pltpu.VMEM_SHARED((ROWS_SP, D), jnp.float32),
          pltpu.SemaphoreType.DMA((NB,)),
          pltpu.SemaphoreType.DMA((NB,)),
      ],
  )
  def seg_sum(h_hbm, src_hbm, dst_hbm, zeros_hbm, out_hbm,
              src2d, dst2d, rows, h_sh, acc_sh, gsem, ssem):
    c = lax.axis_index("c")
    s = lax.axis_index("s")
    wid = c * NS + s

    pltpu.async_copy(src_hbm.at[wid], src2d, gsem.at[0]).wait()
    pltpu.async_copy(dst_hbm.at[wid], dst2d, gsem.at[0]).wait()

    hrows = 624
    hbase = s * hrows
    pltpu.sync_copy(h_hbm.at[pl.ds(hbase, hrows)],
                    h_sh.at[pl.ds(hbase, hrows)])

    @pl.when(s == 0)
    def _h_tail():
      pltpu.sync_copy(h_hbm.at[pl.ds(NS * hrows, N - NS * hrows)],
                      h_sh.at[pl.ds(NS * hrows, N - NS * hrows)])
    zrows = ROWS_SP // NS
    zbase = s * zrows
    pltpu.sync_copy(zeros_hbm.at[pl.ds(zbase, zrows)],
                    acc_sh.at[pl.ds(zbase, zrows)])
    plsc.subcore_barrier()

    def gather(k, b):
      return pltpu.make_async_copy(h_sh.at[src2d.at[k]], rows.at[b],
                                   gsem.at[b])

    def scatter(k, b):
      return pltpu.make_async_copy(rows.at[b], acc_sh.at[dst2d.at[k]],
                                   ssem.at[b])

    for b in range(DA):
      pltpu.async_copy(h_sh.at[src2d.at[b]], rows.at[b], gsem.at[b])

    def body(jj, carry):
      base = jj * NB
      for b in range(NB):
        k = base + b
        gather(k, b).wait()
        pltpu.async_copy(rows.at[b], acc_sh.at[dst2d.at[k]], ssem.at[b],
                         add=True)
        kn = k + DA
        bn = (b + DA) % NB

        @pl.when(jnp.logical_and(kn >= NB, kn < NSTEP))
        def _wait_prev_scatter():
          scatter(kn - NB, bn).wait()

        @pl.when(kn < NSTEP)
        def _fire_ahead():
          pltpu.async_copy(h_sh.at[src2d.at[kn]], rows.at[bn], gsem.at[bn])
      return carry

    lax.fori_loop(0, NSTEP // NB, body, 0)
    for b in range(NB):
      scatter(NSTEP - NB + b, b).wait()
    plsc.subcore_barrier()

    orows = 624
    obase = s * orows
    pltpu.sync_copy(acc_sh.at[pl.ds(obase, orows)],
                    out_hbm.at[c].at[pl.ds(obase, orows)])

    @pl.when(s == 0)
    def _tail():
      pltpu.sync_copy(acc_sh.at[pl.ds(NS * orows, N - NS * orows)],
                      out_hbm.at[c].at[pl.ds(NS * orows, N - NS * orows)])

  return seg_sum


_SEG = _make_seg_sum()


N2 = N // 2


def _dup(v):
  return jnp.concatenate([v, v], axis=1)


def _mm2(z, W, half):
  return jnp.concatenate(
      [jnp.dot(z[:, :half], W, preferred_element_type=jnp.float32),
       jnp.dot(z[:, half:], W, preferred_element_type=jnp.float32)], axis=1)


def _bn_packed(z, g, b, half):
  m = jnp.mean(z, axis=0, keepdims=True)
  q = jnp.mean(z * z, axis=0, keepdims=True)
  mf = (m[:, :half] + m[:, half:]) * 0.5
  qf = (q[:, :half] + q[:, half:]) * 0.5
  var = jnp.maximum(qf - mf * mf, 0.0)
  scale = _dup(g * lax.rsqrt(var + 1e-5))
  shift = _dup(b - g * mf * lax.rsqrt(var + 1e-5))
  return z * scale + shift


def _pre(x, W1):
  dh = W1.shape[1]

  def body(x_ref, w_ref, out_ref):
    out_ref[...] = jnp.dot(x_ref[...], w_ref[...],
                           preferred_element_type=jnp.float32)

  return pl.pallas_call(
      body, out_shape=jax.ShapeDtypeStruct((N, dh), jnp.float32))(x, W1)


def _dense_layer(u2, agg2, b1, g1, be1, W2, b2, gm, bm, W1n):
  has_next = W1n is not None

  def body(*refs):
    (u_ref, agg_ref, b1_ref, g1_ref, be1_ref, w2_ref, b2_ref,
     gm_ref, bm_ref) = refs[:9]
    out_ref = refs[-1]
    z = u_ref[...] + agg_ref[0] + agg_ref[1] + _dup(b1_ref[...])
    z = _bn_packed(z, g1_ref[...], be1_ref[...], D)
    z = jnp.maximum(z, 0.0)
    z = _mm2(z, w2_ref[...], D) + _dup(b2_ref[...])
    z = _bn_packed(z, gm_ref[...], bm_ref[...], D)
    z = jnp.maximum(z, 0.0)
    if has_next:
      z = _mm2(z, refs[9][...], D)
    out_ref[...] = z

  args = [u2, agg2, b1.reshape(1, -1), g1.reshape(1, -1),
          be1.reshape(1, -1), W2, b2.reshape(1, -1),
          gm.reshape(1, -1), bm.reshape(1, -1)]
  if has_next:
    args.append(W1n)
  return pl.pallas_call(
      body,
      out_shape=jax.ShapeDtypeStruct((N2, 2 * D), jnp.float32),
  )(*args)


def _final_layer(h2, agg2, W1, b1, g1, be1, W2, b2, bt_e, bt_o):
  dout = W2.shape[1]

  def body(h_ref, agg_ref, w1_ref, b1_ref, g1_ref, be1_ref,
           w2_ref, b2_ref, bte_ref, bto_ref, out_ref):
    z = h_ref[...] + agg_ref[0] + agg_ref[1]
    z = _mm2(z, w1_ref[...], D) + _dup(b1_ref[...])
    z = _bn_packed(z, g1_ref[...], be1_ref[...], dout)
    z = jnp.maximum(z, 0.0)
    z = _mm2(z, w2_ref[...], dout) + _dup(b2_ref[...])
    oh_e = (lax.broadcasted_iota(jnp.int32, (G, N2), 0) ==
            bte_ref[...]).astype(jnp.float32)
    oh_o = (lax.broadcasted_iota(jnp.int32, (G, N2), 0) ==
            bto_ref[...]).astype(jnp.float32)
    se = jnp.dot(oh_e, z, preferred_element_type=jnp.float32)
    so = jnp.dot(oh_o, z, preferred_element_type=jnp.float32)
    sums = se[:, :dout] + so[:, dout:]
    cnt = jnp.sum(oh_e + oh_o, axis=1, keepdims=True)
    mean = sums / jnp.maximum(cnt, 1.0)
    mx = jnp.max(mean, axis=1, keepdims=True)
    lse = jnp.log(jnp.sum(jnp.exp(mean - mx), axis=1, keepdims=True)) + mx
    out_ref[...] = mean - lse

  return pl.pallas_call(
      body,
      out_shape=jax.ShapeDtypeStruct((G, dout), jnp.float32),
  )(h2, agg2, W1, b1.reshape(1, -1), g1.reshape(1, -1), be1.reshape(1, -1),
    W2, b2.reshape(1, -1), bt_e, bt_o)


def kernel(x, edge_index, batch, params):
  p = list(params)
  layer_p = [p[i * 6:(i + 1) * 6] for i in range(5)]
  norm_p = [p[30 + i * 2:30 + (i + 1) * 2] for i in range(4)]

  src_p = edge_index[0].reshape(NW, CH_PER_TILE, CHUNK)
  dst_p = edge_index[1].reshape(NW, CH_PER_TILE, CHUNK)
  zeros = jnp.zeros((ROWS_SP, D), jnp.float32)

  u2 = _pre(x, layer_p[0][0]).reshape(N2, 2 * D)
  for i in range(4):
    _, b1, g1, be1, W2, b2 = layer_p[i]
    gm, bm = norm_p[i]
    agg = _SEG(u2.reshape(N, D), src_p, dst_p, zeros)
    agg2 = agg.reshape(2, N2, 2 * D)
    W1n = layer_p[i + 1][0] if i < 3 else None
    u2 = _dense_layer(u2, agg2, b1, g1, be1, W2, b2, gm, bm, W1n)

  W1, b1, g1, be1, W2, b2 = layer_p[4]
  agg = _SEG(u2.reshape(N, D), src_p, dst_p, zeros)
  agg2 = agg.reshape(2, N2, 2 * D)
  bt = batch.astype(jnp.int32)
  return _final_layer(u2, agg2, W1, b1, g1, be1, W2, b2,
                      bt[0::2].reshape(1, N2), bt[1::2].reshape(1, N2))


# speedup vs baseline: 17.3807x; 1.0851x over previous
import functools

import jax
import jax.numpy as jnp
from jax import lax
from jax.experimental import pallas as pl
from jax.experimental.pallas import tpu as pltpu
from jax.experimental.pallas import tpu_sc as plsc

N = 10000
E = 320000
G = 64
D = 64

NC = 2
NS = 16
NW = NC * NS

EPT = E // NW
CHUNK = 80
CH_PER_TILE = 125
ROWS_SP = 10240

NB = 5
DA = 2
NSTEP = CH_PER_TILE


def _make_seg_sum():
  mesh = plsc.VectorSubcoreMesh(core_axis_name="c", subcore_axis_name="s")

  @functools.partial(
      pl.kernel,
      mesh=mesh,
      compiler_params=pltpu.CompilerParams(use_tc_tiling_on_sc=False),
      out_type=jax.ShapeDtypeStruct((NC, N, D), jnp.float32),
      scratch_types=[
          pltpu.VMEM((EPT,), jnp.int32),
          pltpu.VMEM((EPT,), jnp.int32),
          pltpu.VMEM((NB, CHUNK, D), jnp.float32),
          pltpu.VMEM_SHARED((N, D), jnp.float32),
          pltpu.VMEM_SHARED((ROWS_SP, D), jnp.float32),
          pltpu.SemaphoreType.DMA((NB,)),
          pltpu.SemaphoreType.DMA((NB,)),
      ],
  )
  def seg_sum(h_hbm, eidx_hbm, zeros_hbm, out_hbm,
              src1d, dst1d, rows, h_sh, acc_sh, gsem, ssem):
    c = lax.axis_index("c")
    s = lax.axis_index("s")
    wid = c * NS + s

    pltpu.async_copy(eidx_hbm.at[0].at[pl.ds(wid * EPT, EPT)], src1d,
                     gsem.at[0]).wait()
    pltpu.async_copy(eidx_hbm.at[1].at[pl.ds(wid * EPT, EPT)], dst1d,
                     gsem.at[0]).wait()

    hrows = 624
    hbase = s * hrows
    pltpu.sync_copy(h_hbm.at[pl.ds(hbase, hrows)],
                    h_sh.at[pl.ds(hbase, hrows)])

    @pl.when(s == 0)
    def _h_tail():
      pltpu.sync_copy(h_hbm.at[pl.ds(NS * hrows, N - NS * hrows)],
                      h_sh.at[pl.ds(NS * hrows, N - NS * hrows)])
    zrows = ROWS_SP // NS
    zbase = s * zrows
    pltpu.sync_copy(zeros_hbm.at[pl.ds(zbase, zrows)],
                    acc_sh.at[pl.ds(zbase, zrows)])
    plsc.subcore_barrier()

    def sidx(k):
      return src1d.at[pl.ds(k * CHUNK, CHUNK)]

    def didx(k):
      return dst1d.at[pl.ds(k * CHUNK, CHUNK)]

    def gather(k, b):
      return pltpu.make_async_copy(h_sh.at[sidx(k)], rows.at[b], gsem.at[b])

    def scatter(k, b):
      return pltpu.make_async_copy(rows.at[b], acc_sh.at[didx(k)],
                                   ssem.at[b])

    for b in range(DA):
      pltpu.async_copy(h_sh.at[sidx(b)], rows.at[b], gsem.at[b])

    def body(jj, carry):
      base = jj * NB
      for b in range(NB):
        k = base + b
        gather(k, b).wait()
        pltpu.async_copy(rows.at[b], acc_sh.at[didx(k)], ssem.at[b],
                         add=True)
        kn = k + DA
        bn = (b + DA) % NB

        @pl.when(jnp.logical_and(kn >= NB, kn < NSTEP))
        def _wait_prev_scatter():
          scatter(kn - NB, bn).wait()

        @pl.when(kn < NSTEP)
        def _fire_ahead():
          pltpu.async_copy(h_sh.at[sidx(kn)], rows.at[bn], gsem.at[bn])
      return carry

    lax.fori_loop(0, NSTEP // NB, body, 0)
    for b in range(NB):
      scatter(NSTEP - NB + b, b).wait()
    plsc.subcore_barrier()

    orows = 624
    obase = s * orows
    pltpu.sync_copy(acc_sh.at[pl.ds(obase, orows)],
                    out_hbm.at[c].at[pl.ds(obase, orows)])

    @pl.when(s == 0)
    def _tail():
      pltpu.sync_copy(acc_sh.at[pl.ds(NS * orows, N - NS * orows)],
                      out_hbm.at[c].at[pl.ds(NS * orows, N - NS * orows)])

  return seg_sum


_SEG = _make_seg_sum()


N2 = N // 2


def _dup(v):
  return jnp.concatenate([v, v], axis=1)


def _mm2(z, W, half):
  return jnp.concatenate(
      [jnp.dot(z[:, :half], W, preferred_element_type=jnp.float32),
       jnp.dot(z[:, half:], W, preferred_element_type=jnp.float32)], axis=1)


def _bn_packed(z, g, b, half):
  m = jnp.mean(z, axis=0, keepdims=True)
  q = jnp.mean(z * z, axis=0, keepdims=True)
  mf = (m[:, :half] + m[:, half:]) * 0.5
  qf = (q[:, :half] + q[:, half:]) * 0.5
  var = jnp.maximum(qf - mf * mf, 0.0)
  scale = _dup(g * lax.rsqrt(var + 1e-5))
  shift = _dup(b - g * mf * lax.rsqrt(var + 1e-5))
  return z * scale + shift


def _pre(x, W1):
  dh = W1.shape[1]

  def body(x_ref, w_ref, out_ref):
    out_ref[...] = jnp.dot(x_ref[...], w_ref[...],
                           preferred_element_type=jnp.float32)

  return pl.pallas_call(
      body, out_shape=jax.ShapeDtypeStruct((N, dh), jnp.float32))(x, W1)


def _dense_layer(u2, agg2, b1, g1, be1, W2, b2, gm, bm, W1n):
  has_next = W1n is not None

  def body(*refs):
    (u_ref, agg_ref, b1_ref, g1_ref, be1_ref, w2_ref, b2_ref,
     gm_ref, bm_ref) = refs[:9]
    out_ref = refs[-1]
    z = u_ref[...] + agg_ref[0] + agg_ref[1] + _dup(b1_ref[...])
    z = _bn_packed(z, g1_ref[...], be1_ref[...], D)
    z = jnp.maximum(z, 0.0)
    z = _mm2(z, w2_ref[...], D) + _dup(b2_ref[...])
    z = _bn_packed(z, gm_ref[...], bm_ref[...], D)
    z = jnp.maximum(z, 0.0)
    if has_next:
      z = _mm2(z, refs[9][...], D)
    out_ref[...] = z

  args = [u2, agg2, b1.reshape(1, -1), g1.reshape(1, -1),
          be1.reshape(1, -1), W2, b2.reshape(1, -1),
          gm.reshape(1, -1), bm.reshape(1, -1)]
  if has_next:
    args.append(W1n)
  return pl.pallas_call(
      body,
      out_shape=jax.ShapeDtypeStruct((N2, 2 * D), jnp.float32),
  )(*args)


def _final_layer(h2, agg2, W1, b1, g1, be1, W2, b2, bt_e, bt_o):
  dout = W2.shape[1]

  def body(h_ref, agg_ref, w1_ref, b1_ref, g1_ref, be1_ref,
           w2_ref, b2_ref, bte_ref, bto_ref, out_ref):
    z = h_ref[...] + agg_ref[0] + agg_ref[1]
    z = _mm2(z, w1_ref[...], D) + _dup(b1_ref[...])
    z = _bn_packed(z, g1_ref[...], be1_ref[...], dout)
    z = jnp.maximum(z, 0.0)
    z = _mm2(z, w2_ref[...], dout) + _dup(b2_ref[...])
    oh_e = (lax.broadcasted_iota(jnp.int32, (G, N2), 0) ==
            bte_ref[...]).astype(jnp.float32)
    oh_o = (lax.broadcasted_iota(jnp.int32, (G, N2), 0) ==
            bto_ref[...]).astype(jnp.float32)
    se = jnp.dot(oh_e, z, preferred_element_type=jnp.float32)
    so = jnp.dot(oh_o, z, preferred_element_type=jnp.float32)
    sums = se[:, :dout] + so[:, dout:]
    cnt = jnp.sum(oh_e + oh_o, axis=1, keepdims=True)
    mean = sums / jnp.maximum(cnt, 1.0)
    mx = jnp.max(mean, axis=1, keepdims=True)
    lse = jnp.log(jnp.sum(jnp.exp(mean - mx), axis=1, keepdims=True)) + mx
    out_ref[...] = mean - lse

  return pl.pallas_call(
      body,
      out_shape=jax.ShapeDtypeStruct((G, dout), jnp.float32),
  )(h2, agg2, W1, b1.reshape(1, -1), g1.reshape(1, -1), be1.reshape(1, -1),
    W2, b2.reshape(1, -1), bt_e, bt_o)


def kernel(x, edge_index, batch, params):
  p = list(params)
  layer_p = [p[i * 6:(i + 1) * 6] for i in range(5)]
  norm_p = [p[30 + i * 2:30 + (i + 1) * 2] for i in range(4)]

  zeros = jnp.zeros((ROWS_SP, D), jnp.float32)

  u2 = _pre(x, layer_p[0][0]).reshape(N2, 2 * D)
  for i in range(4):
    _, b1, g1, be1, W2, b2 = layer_p[i]
    gm, bm = norm_p[i]
    agg = _SEG(u2.reshape(N, D), edge_index, zeros)
    agg2 = agg.reshape(2, N2, 2 * D)
    W1n = layer_p[i + 1][0] if i < 3 else None
    u2 = _dense_layer(u2, agg2, b1, g1, be1, W2, b2, gm, bm, W1n)

  W1, b1, g1, be1, W2, b2 = layer_p[4]
  agg = _SEG(u2.reshape(N, D), edge_index, zeros)
  agg2 = agg.reshape(2, N2, 2 * D)
  bt = batch.astype(jnp.int32)
  return _final_layer(u2, agg2, W1, b1, g1, be1, W2, b2,
                      bt[0::2].reshape(1, N2), bt[1::2].reshape(1, N2))


# speedup vs baseline: 17.4493x; 1.0039x over previous
import functools

import jax
import jax.numpy as jnp
from jax import lax
from jax.experimental import pallas as pl
from jax.experimental.pallas import tpu as pltpu
from jax.experimental.pallas import tpu_sc as plsc

N = 10000
E = 320000
G = 64
D = 64

NC = 2
NS = 16
NW = NC * NS

EPT = E // NW
CHUNK = 80
CH_PER_TILE = 125
ROWS_SP = 10240

NB = 5
DA = 3
NSTEP = CH_PER_TILE


def _make_seg_sum():
  mesh = plsc.VectorSubcoreMesh(core_axis_name="c", subcore_axis_name="s")

  @functools.partial(
      pl.kernel,
      mesh=mesh,
      compiler_params=pltpu.CompilerParams(use_tc_tiling_on_sc=False),
      out_type=jax.ShapeDtypeStruct((NC, N, D), jnp.float32),
      scratch_types=[
          pltpu.VMEM((EPT,), jnp.int32),
          pltpu.VMEM((EPT,), jnp.int32),
          pltpu.VMEM((NB, CHUNK, D), jnp.float32),
          pltpu.VMEM_SHARED((N, D), jnp.float32),
          pltpu.VMEM_SHARED((ROWS_SP, D), jnp.float32),
          pltpu.SemaphoreType.DMA((NB,)),
          pltpu.SemaphoreType.DMA((NB,)),
      ],
  )
  def seg_sum(h_hbm, eidx_hbm, zeros_hbm, out_hbm,
              src1d, dst1d, rows, h_sh, acc_sh, gsem, ssem):
    c = lax.axis_index("c")
    s = lax.axis_index("s")
    wid = c * NS + s

    pltpu.async_copy(eidx_hbm.at[0].at[pl.ds(wid * EPT, EPT)], src1d,
                     gsem.at[0]).wait()
    pltpu.async_copy(eidx_hbm.at[1].at[pl.ds(wid * EPT, EPT)], dst1d,
                     gsem.at[0]).wait()

    hrows = 624
    hbase = s * hrows
    pltpu.sync_copy(h_hbm.at[pl.ds(hbase, hrows)],
                    h_sh.at[pl.ds(hbase, hrows)])

    @pl.when(s == 0)
    def _h_tail():
      pltpu.sync_copy(h_hbm.at[pl.ds(NS * hrows, N - NS * hrows)],
                      h_sh.at[pl.ds(NS * hrows, N - NS * hrows)])
    zrows = ROWS_SP // NS
    zbase = s * zrows
    pltpu.sync_copy(zeros_hbm.at[pl.ds(zbase, zrows)],
                    acc_sh.at[pl.ds(zbase, zrows)])
    plsc.subcore_barrier()

    def sidx(k):
      return src1d.at[pl.ds(k * CHUNK, CHUNK)]

    def didx(k):
      return dst1d.at[pl.ds(k * CHUNK, CHUNK)]

    def gather(k, b):
      return pltpu.make_async_copy(h_sh.at[sidx(k)], rows.at[b], gsem.at[b])

    def scatter(k, b):
      return pltpu.make_async_copy(rows.at[b], acc_sh.at[didx(k)],
                                   ssem.at[b])

    for b in range(DA):
      pltpu.async_copy(h_sh.at[sidx(b)], rows.at[b], gsem.at[b])

    def body(jj, carry):
      base = jj * NB
      for b in range(NB):
        k = base + b
        gather(k, b).wait()
        pltpu.async_copy(rows.at[b], acc_sh.at[didx(k)], ssem.at[b],
                         add=True)
        kn = k + DA
        bn = (b + DA) % NB

        @pl.when(jnp.logical_and(kn >= NB, kn < NSTEP))
        def _wait_prev_scatter():
          scatter(kn - NB, bn).wait()

        @pl.when(kn < NSTEP)
        def _fire_ahead():
          pltpu.async_copy(h_sh.at[sidx(kn)], rows.at[bn], gsem.at[bn])
      return carry

    lax.fori_loop(0, NSTEP // NB, body, 0)
    for b in range(NB):
      scatter(NSTEP - NB + b, b).wait()
    plsc.subcore_barrier()

    orows = 624
    obase = s * orows
    pltpu.sync_copy(acc_sh.at[pl.ds(obase, orows)],
                    out_hbm.at[c].at[pl.ds(obase, orows)])

    @pl.when(s == 0)
    def _tail():
      pltpu.sync_copy(acc_sh.at[pl.ds(NS * orows, N - NS * orows)],
                      out_hbm.at[c].at[pl.ds(NS * orows, N - NS * orows)])

  return seg_sum


_SEG = _make_seg_sum()


N2 = N // 2


def _dup(v):
  return jnp.concatenate([v, v], axis=1)


def _mm2(z, W, half):
  return jnp.concatenate(
      [jnp.dot(z[:, :half], W, preferred_element_type=jnp.float32),
       jnp.dot(z[:, half:], W, preferred_element_type=jnp.float32)], axis=1)


def _bn_packed(z, g, b, half):
  m = jnp.mean(z, axis=0, keepdims=True)
  q = jnp.mean(z * z, axis=0, keepdims=True)
  mf = (m[:, :half] + m[:, half:]) * 0.5
  qf = (q[:, :half] + q[:, half:]) * 0.5
  var = jnp.maximum(qf - mf * mf, 0.0)
  scale = _dup(g * lax.rsqrt(var + 1e-5))
  shift = _dup(b - g * mf * lax.rsqrt(var + 1e-5))
  return z * scale + shift


def _pre(x, W1):
  dh = W1.shape[1]

  def body(x_ref, w_ref, out_ref):
    out_ref[...] = jnp.dot(x_ref[...], w_ref[...],
                           preferred_element_type=jnp.float32)

  return pl.pallas_call(
      body, out_shape=jax.ShapeDtypeStruct((N, dh), jnp.float32))(x, W1)


def _dense_layer(u2, agg2, b1, g1, be1, W2, b2, gm, bm, W1n):
  has_next = W1n is not None

  def body(*refs):
    (u_ref, agg_ref, b1_ref, g1_ref, be1_ref, w2_ref, b2_ref,
     gm_ref, bm_ref) = refs[:9]
    out_ref = refs[-1]
    z = u_ref[...] + agg_ref[0] + agg_ref[1] + _dup(b1_ref[...])
    z = _bn_packed(z, g1_ref[...], be1_ref[...], D)
    z = jnp.maximum(z, 0.0)
    z = _mm2(z, w2_ref[...], D) + _dup(b2_ref[...])
    z = _bn_packed(z, gm_ref[...], bm_ref[...], D)
    z = jnp.maximum(z, 0.0)
    if has_next:
      z = _mm2(z, refs[9][...], D)
    out_ref[...] = z

  args = [u2, agg2, b1.reshape(1, -1), g1.reshape(1, -1),
          be1.reshape(1, -1), W2, b2.reshape(1, -1),
          gm.reshape(1, -1), bm.reshape(1, -1)]
  if has_next:
    args.append(W1n)
  return pl.pallas_call(
      body,
      out_shape=jax.ShapeDtypeStruct((N2, 2 * D), jnp.float32),
  )(*args)


def _final_layer(h2, agg2, W1, b1, g1, be1, W2, b2, bt_e, bt_o):
  dout = W2.shape[1]

  def body(h_ref, agg_ref, w1_ref, b1_ref, g1_ref, be1_ref,
           w2_ref, b2_ref, bte_ref, bto_ref, out_ref):
    z = h_ref[...] + agg_ref[0] + agg_ref[1]
    z = _mm2(z, w1_ref[...], D) + _dup(b1_ref[...])
    z = _bn_packed(z, g1_ref[...], be1_ref[...], dout)
    z = jnp.maximum(z, 0.0)
    z = _mm2(z, w2_ref[...], dout) + _dup(b2_ref[...])
    oh_e = (lax.broadcasted_iota(jnp.int32, (G, N2), 0) ==
            bte_ref[...]).astype(jnp.float32)
    oh_o = (lax.broadcasted_iota(jnp.int32, (G, N2), 0) ==
            bto_ref[...]).astype(jnp.float32)
    se = jnp.dot(oh_e, z, preferred_element_type=jnp.float32)
    so = jnp.dot(oh_o, z, preferred_element_type=jnp.float32)
    sums = se[:, :dout] + so[:, dout:]
    cnt = jnp.sum(oh_e + oh_o, axis=1, keepdims=True)
    mean = sums / jnp.maximum(cnt, 1.0)
    mx = jnp.max(mean, axis=1, keepdims=True)
    lse = jnp.log(jnp.sum(jnp.exp(mean - mx), axis=1, keepdims=True)) + mx
    out_ref[...] = mean - lse

  return pl.pallas_call(
      body,
      out_shape=jax.ShapeDtypeStruct((G, dout), jnp.float32),
  )(h2, agg2, W1, b1.reshape(1, -1), g1.reshape(1, -1), be1.reshape(1, -1),
    W2, b2.reshape(1, -1), bt_e, bt_o)


def kernel(x, edge_index, batch, params):
  p = list(params)
  layer_p = [p[i * 6:(i + 1) * 6] for i in range(5)]
  norm_p = [p[30 + i * 2:30 + (i + 1) * 2] for i in range(4)]

  zeros = jnp.zeros((ROWS_SP, D), jnp.float32)

  u2 = _pre(x, layer_p[0][0]).reshape(N2, 2 * D)
  for i in range(4):
    _, b1, g1, be1, W2, b2 = layer_p[i]
    gm, bm = norm_p[i]
    agg = _SEG(u2.reshape(N, D), edge_index, zeros)
    agg2 = agg.reshape(2, N2, 2 * D)
    W1n = layer_p[i + 1][0] if i < 3 else None
    u2 = _dense_layer(u2, agg2, b1, g1, be1, W2, b2, gm, bm, W1n)

  W1, b1, g1, be1, W2, b2 = layer_p[4]
  agg = _SEG(u2.reshape(N, D), edge_index, zeros)
  agg2 = agg.reshape(2, N2, 2 * D)
  bt = batch.astype(jnp.int32)
  return _final_layer(u2, agg2, W1, b1, g1, be1, W2, b2,
                      bt[0::2].reshape(1, N2), bt[1::2].reshape(1, N2))


# speedup vs baseline: 17.9439x; 1.0283x over previous
import functools

import jax
import jax.numpy as jnp
from jax import lax
from jax.experimental import pallas as pl
from jax.experimental.pallas import tpu as pltpu
from jax.experimental.pallas import tpu_sc as plsc

N = 10000
E = 320000
G = 64
D = 64

NC = 2
NS = 16
NW = NC * NS

EPT = E // NW
CHUNK = 80
CH_PER_TILE = 125
ROWS_SP = 10240

NB = 5
DA = 3
NSTEP = CH_PER_TILE


def _make_seg_sum():
  mesh = plsc.VectorSubcoreMesh(core_axis_name="c", subcore_axis_name="s")

  @functools.partial(
      pl.kernel,
      mesh=mesh,
      compiler_params=pltpu.CompilerParams(use_tc_tiling_on_sc=False),
      out_type=jax.ShapeDtypeStruct((NC, N, D), jnp.float32),
      scratch_types=[
          pltpu.VMEM((EPT,), jnp.int32),
          pltpu.VMEM((EPT,), jnp.int32),
          pltpu.VMEM((NB, CHUNK, D), jnp.float32),
          pltpu.VMEM_SHARED((N, D), jnp.float32),
          pltpu.VMEM_SHARED((ROWS_SP, D), jnp.float32),
          pltpu.SemaphoreType.DMA((NB,)),
          pltpu.SemaphoreType.DMA((NB,)),
      ],
  )
  def seg_sum(h_hbm, eidx_hbm, zeros_hbm, out_hbm,
              src1d, dst1d, rows, h_sh, acc_sh, gsem, ssem):
    c = lax.axis_index("c")
    s = lax.axis_index("s")
    wid = c * NS + s

    hrows = 624
    hbase = s * hrows
    zrows = ROWS_SP // NS
    zbase = s * zrows
    c1 = pltpu.async_copy(eidx_hbm.at[0].at[pl.ds(wid * EPT, EPT)], src1d,
                          gsem.at[0])
    c2 = pltpu.async_copy(eidx_hbm.at[1].at[pl.ds(wid * EPT, EPT)], dst1d,
                          gsem.at[1])
    c3 = pltpu.async_copy(h_hbm.at[pl.ds(hbase, hrows)],
                          h_sh.at[pl.ds(hbase, hrows)], gsem.at[2])
    c4 = pltpu.async_copy(zeros_hbm.at[pl.ds(zbase, zrows)],
                          acc_sh.at[pl.ds(zbase, zrows)], gsem.at[3])

    @pl.when(s == 0)
    def _h_tail():
      pltpu.async_copy(h_hbm.at[pl.ds(NS * hrows, N - NS * hrows)],
                       h_sh.at[pl.ds(NS * hrows, N - NS * hrows)],
                       gsem.at[4]).wait()

    c1.wait()
    c2.wait()
    c3.wait()
    c4.wait()
    plsc.subcore_barrier()

    def sidx(k):
      return src1d.at[pl.ds(k * CHUNK, CHUNK)]

    def didx(k):
      return dst1d.at[pl.ds(k * CHUNK, CHUNK)]

    def gather(k, b):
      return pltpu.make_async_copy(h_sh.at[sidx(k)], rows.at[b], gsem.at[b])

    def scatter(k, b):
      return pltpu.make_async_copy(rows.at[b], acc_sh.at[didx(k)],
                                   ssem.at[b])

    for b in range(DA):
      pltpu.async_copy(h_sh.at[sidx(b)], rows.at[b], gsem.at[b])

    def body(jj, carry):
      base = jj * NB
      for b in range(NB):
        k = base + b
        gather(k, b).wait()
        pltpu.async_copy(rows.at[b], acc_sh.at[didx(k)], ssem.at[b],
                         add=True)
        kn = k + DA
        bn = (b + DA) % NB

        @pl.when(jnp.logical_and(kn >= NB, kn < NSTEP))
        def _wait_prev_scatter():
          scatter(kn - NB, bn).wait()

        @pl.when(kn < NSTEP)
        def _fire_ahead():
          pltpu.async_copy(h_sh.at[sidx(kn)], rows.at[bn], gsem.at[bn])
      return carry

    lax.fori_loop(0, NSTEP // NB, body, 0)
    for b in range(NB):
      scatter(NSTEP - NB + b, b).wait()
    plsc.subcore_barrier()

    orows = 624
    obase = s * orows
    pltpu.sync_copy(acc_sh.at[pl.ds(obase, orows)],
                    out_hbm.at[c].at[pl.ds(obase, orows)])

    @pl.when(s == 0)
    def _tail():
      pltpu.sync_copy(acc_sh.at[pl.ds(NS * orows, N - NS * orows)],
                      out_hbm.at[c].at[pl.ds(NS * orows, N - NS * orows)])

  return seg_sum


_SEG = _make_seg_sum()


N2 = N // 2


def _dup(v):
  return jnp.concatenate([v, v], axis=1)


def _mm2(z, W, half):
  return jnp.concatenate(
      [jnp.dot(z[:, :half], W, preferred_element_type=jnp.float32),
       jnp.dot(z[:, half:], W, preferred_element_type=jnp.float32)], axis=1)


def _bn_packed(z, g, b, half):
  m = jnp.mean(z, axis=0, keepdims=True)
  q = jnp.mean(z * z, axis=0, keepdims=True)
  mf = (m[:, :half] + m[:, half:]) * 0.5
  qf = (q[:, :half] + q[:, half:]) * 0.5
  var = jnp.maximum(qf - mf * mf, 0.0)
  scale = _dup(g * lax.rsqrt(var + 1e-5))
  shift = _dup(b - g * mf * lax.rsqrt(var + 1e-5))
  return z * scale + shift


def _pre(x, W1):
  dh = W1.shape[1]

  def body(x_ref, w_ref, out_ref):
    out_ref[...] = jnp.dot(x_ref[...], w_ref[...],
                           preferred_element_type=jnp.float32)

  return pl.pallas_call(
      body, out_shape=jax.ShapeDtypeStruct((N, dh), jnp.float32))(x, W1)


def _dense_layer(u2, agg2, b1, g1, be1, W2, b2, gm, bm, W1n):
  has_next = W1n is not None

  def body(*refs):
    (u_ref, agg_ref, b1_ref, g1_ref, be1_ref, w2_ref, b2_ref,
     gm_ref, bm_ref) = refs[:9]
    out_ref = refs[-1]
    z = u_ref[...] + agg_ref[0] + agg_ref[1] + _dup(b1_ref[...])
    z = _bn_packed(z, g1_ref[...], be1_ref[...], D)
    z = jnp.maximum(z, 0.0)
    z = _mm2(z, w2_ref[...], D) + _dup(b2_ref[...])
    z = _bn_packed(z, gm_ref[...], bm_ref[...], D)
    z = jnp.maximum(z, 0.0)
    if has_next:
      z = _mm2(z, refs[9][...], D)
    out_ref[...] = z

  args = [u2, agg2, b1.reshape(1, -1), g1.reshape(1, -1),
          be1.reshape(1, -1), W2, b2.reshape(1, -1),
          gm.reshape(1, -1), bm.reshape(1, -1)]
  if has_next:
    args.append(W1n)
  return pl.pallas_call(
      body,
      out_shape=jax.ShapeDtypeStruct((N2, 2 * D), jnp.float32),
  )(*args)


def _final_layer(h2, agg2, W1, b1, g1, be1, W2, b2, bt_e, bt_o):
  dout = W2.shape[1]

  def body(h_ref, agg_ref, w1_ref, b1_ref, g1_ref, be1_ref,
           w2_ref, b2_ref, bte_ref, bto_ref, out_ref):
    z = h_ref[...] + agg_ref[0] + agg_ref[1]
    z = _mm2(z, w1_ref[...], D) + _dup(b1_ref[...])
    z = _bn_packed(z, g1_ref[...], be1_ref[...], dout)
    z = jnp.maximum(z, 0.0)
    z = _mm2(z, w2_ref[...], dout) + _dup(b2_ref[...])
    oh_e = (lax.broadcasted_iota(jnp.int32, (G, N2), 0) ==
            bte_ref[...]).astype(jnp.float32)
    oh_o = (lax.broadcasted_iota(jnp.int32, (G, N2), 0) ==
            bto_ref[...]).astype(jnp.float32)
    se = jnp.dot(oh_e, z, preferred_element_type=jnp.float32)
    so = jnp.dot(oh_o, z, preferred_element_type=jnp.float32)
    sums = se[:, :dout] + so[:, dout:]
    cnt = jnp.sum(oh_e + oh_o, axis=1, keepdims=True)
    mean = sums / jnp.maximum(cnt, 1.0)
    mx = jnp.max(mean, axis=1, keepdims=True)
    lse = jnp.log(jnp.sum(jnp.exp(mean - mx), axis=1, keepdims=True)) + mx
    out_ref[...] = mean - lse

  return pl.pallas_call(
      body,
      out_shape=jax.ShapeDtypeStruct((G, dout), jnp.float32),
  )(h2, agg2, W1, b1.reshape(1, -1), g1.reshape(1, -1), be1.reshape(1, -1),
    W2, b2.reshape(1, -1), bt_e, bt_o)


def kernel(x, edge_index, batch, params):
  p = list(params)
  layer_p = [p[i * 6:(i + 1) * 6] for i in range(5)]
  norm_p = [p[30 + i * 2:30 + (i + 1) * 2] for i in range(4)]

  zeros = jnp.zeros((ROWS_SP, D), jnp.float32)

  u2 = _pre(x, layer_p[0][0]).reshape(N2, 2 * D)
  for i in range(4):
    _, b1, g1, be1, W2, b2 = layer_p[i]
    gm, bm = norm_p[i]
    agg = _SEG(u2.reshape(N, D), edge_index, zeros)
    agg2 = agg.reshape(2, N2, 2 * D)
    W1n = layer_p[i + 1][0] if i < 3 else None
    u2 = _dense_layer(u2, agg2, b1, g1, be1, W2, b2, gm, bm, W1n)

  W1, b1, g1, be1, W2, b2 = layer_p[4]
  agg = _SEG(u2.reshape(N, D), edge_index, zeros)
  agg2 = agg.reshape(2, N2, 2 * D)
  bt = batch.astype(jnp.int32)
  return _final_layer(u2, agg2, W1, b1, g1, be1, W2, b2,
                      bt[0::2].reshape(1, N2), bt[1::2].reshape(1, N2))
